# Initial kernel scaffold; baseline (speedup 1.0000x reference)
import jax
import jax.numpy as jnp
from jax.experimental import pallas as pl


def kernel(x, edge_index, Wl0, bl0, Wr0, Wl1, bl1, Wr1, Wl2, bl2, Wr2):
    raise NotImplementedError("write your pallas kernel here")


# speedup vs baseline: 2.9649x; 2.9649x over previous
import functools

import jax
import jax.numpy as jnp
from jax import lax
from jax.experimental import pallas as pl
from jax.experimental.pallas import tpu as pltpu
from jax.experimental.pallas import tpu_sc as plsc

_N = 10000
_E = 320000
_D = 32
_NC = 2
_NS = 16
_NW = _NC * _NS
_EW = _E // _NW
_C = 128
_NFULL = _EW // _C
_TAIL = _EW - _NFULL * _C
_RPT = 624
_EXTRA_START = _RPT * _NS
_EXTRA = _N - _EXTRA_START


def _make_agg(nparts):
  mesh = plsc.VectorSubcoreMesh(core_axis_name="c", subcore_axis_name="s")
  out_type = [jax.ShapeDtypeStruct((_NC * _N, _D), jnp.float32)
              for _ in range(nparts)]
  scratch = [
      pltpu.VMEM((_C,), jnp.int32),
      pltpu.VMEM((_C,), jnp.int32),
      pltpu.VMEM((_C, _D), jnp.float32),
      pltpu.VMEM((_TAIL,), jnp.int32),
      pltpu.VMEM((_TAIL,), jnp.int32),
      pltpu.VMEM((_TAIL, _D), jnp.float32),
      pltpu.VMEM((_RPT + _EXTRA, _D), jnp.float32),
      pltpu.VMEM((_RPT + _EXTRA, _D), jnp.float32),
      pltpu.VMEM_SHARED((_N, _D), jnp.float32),
      pltpu.SemaphoreType.DMA,
  ]

  def body(*refs):
    ys = refs[:nparts]
    src_hbm, dst_hbm = refs[nparts:nparts + 2]
    outs = refs[nparts + 2:2 * nparts + 2]
    (sidx, didx, rows, sidx_t, didx_t, rows_t, zbuf, rbuf, acc,
     sem) = refs[2 * nparts + 2:]
    cid = lax.axis_index("c")
    sid = lax.axis_index("s")
    wid = sid * _NC + cid
    base = wid * _EW

    def _zrow(i, carry):
      for k in range(_D // 16):
        zbuf[i, pl.ds(k * 16, 16)] = jnp.zeros((16,), jnp.float32)
      return carry
    lax.fori_loop(0, _RPT + _EXTRA, _zrow, 0)

    for h in range(nparts):
      y_hbm = ys[h]
      out_hbm = outs[h]

      pltpu.sync_copy(zbuf.at[pl.ds(0, _RPT)],
                      acc.at[pl.ds(sid * _RPT, _RPT)])

      @pl.when(sid == _NS - 1)
      def _():
        pltpu.sync_copy(zbuf.at[pl.ds(0, _EXTRA)],
                        acc.at[pl.ds(_EXTRA_START, _EXTRA)])

      plsc.subcore_barrier()

      def step(j, carry):
        off = pl.multiple_of(base + j * _C, 8)
        pltpu.sync_copy(src_hbm.at[pl.ds(off, _C)], sidx)
        pltpu.sync_copy(dst_hbm.at[pl.ds(off, _C)], didx)
        pltpu.async_copy(y_hbm.at[sidx], rows, sem).wait()
        pltpu.sync_copy(rows, acc.at[didx], add=True)
        return carry

      lax.fori_loop(0, _NFULL, step, 0)

      offt = pl.multiple_of(base + _NFULL * _C, 8)
      pltpu.sync_copy(src_hbm.at[pl.ds(offt, _TAIL)], sidx_t)
      pltpu.sync_copy(dst_hbm.at[pl.ds(offt, _TAIL)], didx_t)
      pltpu.async_copy(y_hbm.at[sidx_t], rows_t, sem).wait()
      pltpu.sync_copy(rows_t, acc.at[didx_t], add=True)

      plsc.subcore_barrier()

      rs = sid * _RPT
      pltpu.sync_copy(acc.at[pl.ds(rs, _RPT)], rbuf.at[pl.ds(0, _RPT)])
      pltpu.sync_copy(rbuf.at[pl.ds(0, _RPT)],
                      out_hbm.at[pl.ds(cid * _N + rs, _RPT)])

      @pl.when(sid == _NS - 1)
      def _():
        pltpu.sync_copy(acc.at[pl.ds(_EXTRA_START, _EXTRA)], rows_t)
        pltpu.sync_copy(rows_t,
                        out_hbm.at[pl.ds(cid * _N + _EXTRA_START, _EXTRA)])

      if h + 1 < nparts:
        plsc.subcore_barrier()

  return pl.kernel(
      body, out_type=out_type, mesh=mesh, scratch_types=scratch,
      compiler_params=pltpu.CompilerParams(use_tc_tiling_on_sc=False))


def _make_cnt():
  mesh = plsc.VectorSubcoreMesh(core_axis_name="c", subcore_axis_name="s")
  scratch = [
      pltpu.VMEM((_C,), jnp.int32),
      pltpu.VMEM((_TAIL,), jnp.int32),
      pltpu.VMEM((_C,), jnp.float32),
      pltpu.VMEM((_TAIL,), jnp.float32),
      pltpu.VMEM((_RPT + _EXTRA,), jnp.float32),
      pltpu.VMEM_SHARED((_N,), jnp.float32),
  ]

  def body(dst_hbm, cnt_hbm, didx, didx_t, ones, ones_t, cbuf, cacc):
    cid = lax.axis_index("c")
    sid = lax.axis_index("s")
    wid = sid * _NC + cid
    base = wid * _EW

    for k in range(_C // 16):
      ones[pl.ds(k * 16, 16)] = jnp.ones((16,), jnp.float32)
    ones_t[pl.ds(0, 16)] = jnp.ones((16,), jnp.float32)
    for k in range((_RPT + _EXTRA) // 16):
      cbuf[pl.ds(k * 16, 16)] = jnp.zeros((16,), jnp.float32)
    pltpu.sync_copy(cbuf.at[pl.ds(0, _RPT)],
                    cacc.at[pl.ds(sid * _RPT, _RPT)])

    @pl.when(sid == _NS - 1)
    def _():
      pltpu.sync_copy(cbuf.at[pl.ds(0, _EXTRA)],
                      cacc.at[pl.ds(_EXTRA_START, _EXTRA)])

    plsc.subcore_barrier()

    def step(j, carry):
      off = pl.multiple_of(base + j * _C, 8)
      pltpu.sync_copy(dst_hbm.at[pl.ds(off, _C)], didx)
      pltpu.sync_copy(ones, cacc.at[didx], add=True)
      return carry

    lax.fori_loop(0, _NFULL, step, 0)

    offt = pl.multiple_of(base + _NFULL * _C, 8)
    pltpu.sync_copy(dst_hbm.at[pl.ds(offt, _TAIL)], didx_t)
    pltpu.sync_copy(ones_t, cacc.at[didx_t], add=True)

    plsc.subcore_barrier()

    rs = sid * _RPT
    pltpu.sync_copy(cacc.at[pl.ds(rs, _RPT)], cbuf.at[pl.ds(0, _RPT)])
    pltpu.sync_copy(cbuf.at[pl.ds(0, _RPT)],
                    cnt_hbm.at[pl.ds(cid * _N + rs, _RPT)])

    @pl.when(sid == _NS - 1)
    def _():
      pltpu.sync_copy(cacc.at[pl.ds(_EXTRA_START, _EXTRA)], ones_t)
      pltpu.sync_copy(ones_t,
                      cnt_hbm.at[pl.ds(cid * _N + _EXTRA_START, _EXTRA)])

  return pl.kernel(
      body, out_type=[jax.ShapeDtypeStruct((_NC * _N,), jnp.float32)],
      mesh=mesh, scratch_types=scratch,
      compiler_params=pltpu.CompilerParams(use_tc_tiling_on_sc=False))


_agg4 = _make_agg(4)
_agg2 = _make_agg(2)
_cnt = _make_cnt()


_R = 1000
_G = _N // _R


def _s0_body(x_ref, w_ref, *out_refs, splits):
  res = jnp.dot(x_ref[...], w_ref[...], preferred_element_type=jnp.float32)
  off = 0
  for r, s in zip(out_refs, splits):
    r[...] = res[:, off:off + s]
    off += s


def _stage0(x, wcat, splits):
  K = x.shape[1]
  return pl.pallas_call(
      functools.partial(_s0_body, splits=splits),
      grid=(_G,),
      in_specs=[pl.BlockSpec((_R, K), lambda i: (i, 0)),
                pl.BlockSpec((K, sum(splits)), lambda i: (0, 0))],
      out_specs=[pl.BlockSpec((_R, s), lambda i: (i, 0)) for s in splits],
      out_shape=[jax.ShapeDtypeStruct((_N, s), jnp.float32) for s in splits],
  )(x, wcat)


def _mid_body(*refs, nq, splits):
  part_refs = refs[:nq]
  cnt_ref, p_ref, b_ref, w_ref = refs[nq:nq + 4]
  out_refs = refs[nq + 4:]
  cnt = cnt_ref[0] + cnt_ref[1]
  inv = 1.0 / jnp.maximum(cnt, 1.0)
  mean = jnp.concatenate([(pr[0] + pr[1]) * inv for pr in part_refs], axis=1)
  h = jnp.maximum(mean + b_ref[...] + p_ref[...], 0.0)
  res = jnp.dot(h, w_ref[...], preferred_element_type=jnp.float32)
  off = 0
  for r, s in zip(out_refs, splits):
    r[...] = res[:, off:off + s]
    off += s


def _mid(parts_list, cnt3, p, b, wcat, splits):
  nq = len(parts_list)
  Din = nq * _D
  return pl.pallas_call(
      functools.partial(_mid_body, nq=nq, splits=splits),
      grid=(_G,),
      in_specs=([pl.BlockSpec((_NC, _R, _D), lambda i: (0, i, 0))] * nq
                + [pl.BlockSpec((_NC, _R, 1), lambda i: (0, i, 0)),
                   pl.BlockSpec((_R, Din), lambda i: (i, 0)),
                   pl.BlockSpec((1, Din), lambda i: (0, 0)),
                   pl.BlockSpec((Din, sum(splits)), lambda i: (0, 0))]),
      out_specs=[pl.BlockSpec((_R, s), lambda i: (i, 0)) for s in splits],
      out_shape=[jax.ShapeDtypeStruct((_N, s), jnp.float32) for s in splits],
  )(*parts_list, cnt3, p, b, wcat)


def _final_body(pa_ref, pb_ref, cnt_ref, p_ref, b_ref, o_ref):
  cnt = cnt_ref[0] + cnt_ref[1]
  inv = 1.0 / jnp.maximum(cnt, 1.0)
  mean = jnp.concatenate([(pa_ref[0] + pa_ref[1]) * inv,
                          (pb_ref[0] + pb_ref[1]) * inv], axis=1)
  o = mean + b_ref[...] + p_ref[...]
  m = jnp.max(o, axis=1, keepdims=True)
  lse = jnp.log(jnp.sum(jnp.exp(o - m), axis=1, keepdims=True))
  o_ref[...] = o - m - lse


def _final(parts_a, parts_b, cnt3, p, b):
  D = 2 * _D
  return pl.pallas_call(
      _final_body,
      grid=(_G,),
      in_specs=[pl.BlockSpec((_NC, _R, _D), lambda i: (0, i, 0)),
                pl.BlockSpec((_NC, _R, _D), lambda i: (0, i, 0)),
                pl.BlockSpec((_NC, _R, 1), lambda i: (0, i, 0)),
                pl.BlockSpec((_R, D), lambda i: (i, 0)),
                pl.BlockSpec((1, D), lambda i: (0, 0))],
      out_specs=pl.BlockSpec((_R, D), lambda i: (i, 0)),
      out_shape=jax.ShapeDtypeStruct((_N, D), jnp.float32),
  )(parts_a, parts_b, cnt3, p, b)


def kernel(x, edge_index, Wl0, bl0, Wr0, Wl1, bl1, Wr1, Wl2, bl2, Wr2):
  src = edge_index[0]
  dst = edge_index[1]
  w0 = jnp.concatenate([Wl0, Wr0], axis=1)
  w1 = jnp.concatenate([Wl1, Wr1], axis=1)
  w2 = jnp.concatenate([Wl2, Wr2], axis=1)
  b0 = bl0.reshape(1, -1)
  b1 = bl1.reshape(1, -1)
  b2 = bl2.reshape(1, -1)

  sp = (_D, _D, _D, _D)

  y0 = _stage0(x, w0, sp + (128,))
  cnt_flat = _cnt(dst)[0]
  cnt3 = cnt_flat.reshape(_NC, _N, 1)
  parts0 = _agg4(y0[0], y0[1], y0[2], y0[3], src, dst)
  parts0 = [q.reshape(_NC, _N, _D) for q in parts0]

  y1 = _mid(parts0, cnt3, y0[4], b0, w1, sp + (128,))
  parts1 = _agg4(y1[0], y1[1], y1[2], y1[3], src, dst)
  parts1 = [q.reshape(_NC, _N, _D) for q in parts1]

  y2 = _mid(parts1, cnt3, y1[4], b1, w2, (_D, _D, 64))
  parts2 = _agg2(y2[0], y2[1], src, dst)
  parts2 = [q.reshape(_NC, _N, _D) for q in parts2]

  return _final(parts2[0], parts2[1], cnt3, y2[2], b2)


# speedup vs baseline: 5.8977x; 1.9891x over previous
import functools

import jax
import jax.numpy as jnp
from jax import lax
from jax.experimental import pallas as pl
from jax.experimental.pallas import tpu as pltpu
from jax.experimental.pallas import tpu_sc as plsc

_N = 10000
_E = 320000
_D = 32
_NC = 2
_NS = 16
_NW = _NC * _NS
_C = 128
_ROWS = _E // _C
_CPW = _ROWS // _NW
_NPAIR = _CPW // 2
_XTRA = _ROWS - _CPW * _NW
_RPT = 624
_EXTRA_START = _RPT * _NS
_EXTRA = _N - _EXTRA_START


def _make_agg(nparts, with_cnt):
  mesh = plsc.VectorSubcoreMesh(core_axis_name="c", subcore_axis_name="s")
  out_type = [jax.ShapeDtypeStruct((_NC * _N, _D), jnp.float32)
              for _ in range(nparts)]
  if with_cnt:
    out_type.append(jax.ShapeDtypeStruct((_NC * _N,), jnp.float32))
  scratch = [
      pltpu.VMEM((_CPW, _C), jnp.int32),
      pltpu.VMEM((_CPW, _C), jnp.int32),
      pltpu.VMEM((1, _C), jnp.int32),
      pltpu.VMEM((1, _C), jnp.int32),
      pltpu.VMEM((_C, _D), jnp.float32),
      pltpu.VMEM((_C, _D), jnp.float32),
      pltpu.VMEM((_RPT + _EXTRA, _D), jnp.float32),
      pltpu.VMEM((_RPT + _EXTRA, _D), jnp.float32),
      pltpu.VMEM_SHARED((_N, _D), jnp.float32),
      pltpu.SemaphoreType.DMA,
      pltpu.SemaphoreType.DMA,
      pltpu.SemaphoreType.DMA,
      pltpu.SemaphoreType.DMA,
  ]
  if with_cnt:
    scratch += [
        pltpu.VMEM((_C,), jnp.float32),
        pltpu.VMEM((_RPT + _EXTRA,), jnp.float32),
        pltpu.VMEM_SHARED((_N,), jnp.float32),
        pltpu.SemaphoreType.DMA,
    ]

  def body(*refs):
    ys = refs[:nparts]
    srcr_hbm, dstr_hbm = refs[nparts:nparts + 2]
    outs = refs[nparts + 2:2 * nparts + 2]
    rest = refs[2 * nparts + 2:]
    if with_cnt:
      (cnt_hbm, sidx2, didx2, sidxe, didxe, rowsA, rowsB, zbuf, rbuf, acc,
       semGA, semGB, semSA, semSB, ones, cbuf, cacc, semC) = rest
    else:
      (sidx2, didx2, sidxe, didxe, rowsA, rowsB, zbuf, rbuf, acc,
       semGA, semGB, semSA, semSB) = rest
    cid = lax.axis_index("c")
    sid = lax.axis_index("s")
    wid = sid * _NC + cid

    pltpu.sync_copy(srcr_hbm.at[pl.ds(wid * _CPW, _CPW)], sidx2)
    pltpu.sync_copy(dstr_hbm.at[pl.ds(wid * _CPW, _CPW)], didx2)

    @pl.when(wid < _XTRA)
    def _():
      pltpu.sync_copy(srcr_hbm.at[pl.ds(_CPW * _NW + wid, 1)], sidxe)
      pltpu.sync_copy(dstr_hbm.at[pl.ds(_CPW * _NW + wid, 1)], didxe)

    def _zrow(i, carry):
      for k in range(_D // 16):
        zbuf[i, pl.ds(k * 16, 16)] = jnp.zeros((16,), jnp.float32)
      return carry
    lax.fori_loop(0, _RPT + _EXTRA, _zrow, 0)

    if with_cnt:
      for k in range(_C // 16):
        ones[pl.ds(k * 16, 16)] = jnp.ones((16,), jnp.float32)
      for k in range((_RPT + _EXTRA) // 16):
        cbuf[pl.ds(k * 16, 16)] = jnp.zeros((16,), jnp.float32)
      pltpu.sync_copy(cbuf.at[pl.ds(0, _RPT)],
                      cacc.at[pl.ds(sid * _RPT, _RPT)])

      @pl.when(sid == _NS - 1)
      def _():
        pltpu.sync_copy(cbuf.at[pl.ds(0, _EXTRA)],
                        cacc.at[pl.ds(_EXTRA_START, _EXTRA)])

    for h in range(nparts):
      y_hbm = ys[h]
      out_hbm = outs[h]
      do_cnt = with_cnt and h == 0

      pltpu.sync_copy(zbuf.at[pl.ds(0, _RPT)],
                      acc.at[pl.ds(sid * _RPT, _RPT)])

      @pl.when(sid == _NS - 1)
      def _():
        pltpu.sync_copy(zbuf.at[pl.ds(0, _EXTRA)],
                        acc.at[pl.ds(_EXTRA_START, _EXTRA)])

      plsc.subcore_barrier()

      pltpu.async_copy(y_hbm.at[sidx2.at[0]], rowsA, semGA)

      def pair(j2, carry):
        ja = j2 * 2
        jb = ja + 1
        pltpu.make_async_copy(y_hbm.at[sidx2.at[ja]], rowsA, semGA).wait()

        @pl.when(j2 > 0)
        def _():
          pltpu.make_async_copy(rowsB, acc.at[didx2.at[0]], semSB).wait()

        pltpu.async_copy(y_hbm.at[sidx2.at[jb]], rowsB, semGB)
        pltpu.async_copy(rowsA, acc.at[didx2.at[ja]], semSA, add=True)
        if do_cnt:
          pltpu.async_copy(ones, cacc.at[didx2.at[ja]], semC, add=True)
          pltpu.async_copy(ones, cacc.at[didx2.at[jb]], semC, add=True)
        pltpu.make_async_copy(y_hbm.at[sidx2.at[jb]], rowsB, semGB).wait()
        pltpu.make_async_copy(rowsA, acc.at[didx2.at[ja]], semSA).wait()
        pltpu.async_copy(rowsB, acc.at[didx2.at[jb]], semSB, add=True)

        @pl.when(j2 < _NPAIR - 1)
        def _():
          pltpu.async_copy(y_hbm.at[sidx2.at[ja + 2]], rowsA, semGA)

        return carry

      lax.fori_loop(0, _NPAIR, pair, 0)
      pltpu.make_async_copy(rowsB, acc.at[didx2.at[0]], semSB).wait()

      @pl.when(wid < _XTRA)
      def _():
        pltpu.async_copy(y_hbm.at[sidxe.at[0]], rowsA, semGA).wait()
        pltpu.sync_copy(rowsA, acc.at[didxe.at[0]], add=True)
        if do_cnt:
          pltpu.sync_copy(ones, cacc.at[didxe.at[0]], add=True)

      if do_cnt:
        def drain(i, carry):
          pltpu.make_async_copy(ones, cacc.at[didx2.at[0]], semC).wait()
          return carry
        lax.fori_loop(0, _CPW, drain, 0)

      plsc.subcore_barrier()

      rs = sid * _RPT
      pltpu.sync_copy(acc.at[pl.ds(rs, _RPT)], rbuf.at[pl.ds(0, _RPT)])
      pltpu.sync_copy(rbuf.at[pl.ds(0, _RPT)],
                      out_hbm.at[pl.ds(cid * _N + rs, _RPT)])
      if do_cnt:
        pltpu.sync_copy(cacc.at[pl.ds(rs, _RPT)], cbuf.at[pl.ds(0, _RPT)])
        pltpu.sync_copy(cbuf.at[pl.ds(0, _RPT)],
                        cnt_hbm.at[pl.ds(cid * _N + rs, _RPT)])

      @pl.when(sid == _NS - 1)
      def _():
        pltpu.sync_copy(acc.at[pl.ds(_EXTRA_START, _EXTRA)],
                        rbuf.at[pl.ds(_RPT, _EXTRA)])
        pltpu.sync_copy(rbuf.at[pl.ds(_RPT, _EXTRA)],
                        out_hbm.at[pl.ds(cid * _N + _EXTRA_START, _EXTRA)])
        if do_cnt:
          pltpu.sync_copy(cacc.at[pl.ds(_EXTRA_START, _EXTRA)],
                          cbuf.at[pl.ds(_RPT, _EXTRA)])
          pltpu.sync_copy(cbuf.at[pl.ds(_RPT, _EXTRA)],
                          cnt_hbm.at[pl.ds(cid * _N + _EXTRA_START, _EXTRA)])

      if h + 1 < nparts:
        plsc.subcore_barrier()

  return pl.kernel(
      body, out_type=out_type, mesh=mesh, scratch_types=scratch,
      compiler_params=pltpu.CompilerParams(use_tc_tiling_on_sc=False))


_agg4c = _make_agg(4, True)
_agg4 = _make_agg(4, False)
_agg2 = _make_agg(2, False)


_R = 1000
_G = _N // _R


def _s0_body(x_ref, w_ref, *out_refs, splits):
  res = jnp.dot(x_ref[...], w_ref[...], preferred_element_type=jnp.float32)
  off = 0
  for r, s in zip(out_refs, splits):
    r[...] = res[:, off:off + s]
    off += s


def _stage0(x, wcat, splits):
  K = x.shape[1]
  return pl.pallas_call(
      functools.partial(_s0_body, splits=splits),
      grid=(_G,),
      in_specs=[pl.BlockSpec((_R, K), lambda i: (i, 0)),
                pl.BlockSpec((K, sum(splits)), lambda i: (0, 0))],
      out_specs=[pl.BlockSpec((_R, s), lambda i: (i, 0)) for s in splits],
      out_shape=[jax.ShapeDtypeStruct((_N, s), jnp.float32) for s in splits],
  )(x, wcat)


def _mid_body(*refs, nq, splits):
  part_refs = refs[:nq]
  cnt_ref, p_ref, b_ref, w_ref = refs[nq:nq + 4]
  out_refs = refs[nq + 4:]
  cnt = cnt_ref[0] + cnt_ref[1]
  inv = 1.0 / jnp.maximum(cnt, 1.0)
  mean = jnp.concatenate([(pr[0] + pr[1]) * inv for pr in part_refs], axis=1)
  h = jnp.maximum(mean + b_ref[...] + p_ref[...], 0.0)
  res = jnp.dot(h, w_ref[...], preferred_element_type=jnp.float32)
  off = 0
  for r, s in zip(out_refs, splits):
    r[...] = res[:, off:off + s]
    off += s


def _mid(parts_list, cnt3, p, b, wcat, splits):
  nq = len(parts_list)
  Din = nq * _D
  return pl.pallas_call(
      functools.partial(_mid_body, nq=nq, splits=splits),
      grid=(_G,),
      in_specs=([pl.BlockSpec((_NC, _R, _D), lambda i: (0, i, 0))] * nq
                + [pl.BlockSpec((_NC, _R, 1), lambda i: (0, i, 0)),
                   pl.BlockSpec((_R, Din), lambda i: (i, 0)),
                   pl.BlockSpec((1, Din), lambda i: (0, 0)),
                   pl.BlockSpec((Din, sum(splits)), lambda i: (0, 0))]),
      out_specs=[pl.BlockSpec((_R, s), lambda i: (i, 0)) for s in splits],
      out_shape=[jax.ShapeDtypeStruct((_N, s), jnp.float32) for s in splits],
  )(*parts_list, cnt3, p, b, wcat)


def _final_body(pa_ref, pb_ref, cnt_ref, p_ref, b_ref, o_ref):
  cnt = cnt_ref[0] + cnt_ref[1]
  inv = 1.0 / jnp.maximum(cnt, 1.0)
  mean = jnp.concatenate([(pa_ref[0] + pa_ref[1]) * inv,
                          (pb_ref[0] + pb_ref[1]) * inv], axis=1)
  o = mean + b_ref[...] + p_ref[...]
  m = jnp.max(o, axis=1, keepdims=True)
  lse = jnp.log(jnp.sum(jnp.exp(o - m), axis=1, keepdims=True))
  o_ref[...] = o - m - lse


def _final(parts_a, parts_b, cnt3, p, b):
  D = 2 * _D
  return pl.pallas_call(
      _final_body,
      grid=(_G,),
      in_specs=[pl.BlockSpec((_NC, _R, _D), lambda i: (0, i, 0)),
                pl.BlockSpec((_NC, _R, _D), lambda i: (0, i, 0)),
                pl.BlockSpec((_NC, _R, 1), lambda i: (0, i, 0)),
                pl.BlockSpec((_R, D), lambda i: (i, 0)),
                pl.BlockSpec((1, D), lambda i: (0, 0))],
      out_specs=pl.BlockSpec((_R, D), lambda i: (i, 0)),
      out_shape=jax.ShapeDtypeStruct((_N, D), jnp.float32),
  )(parts_a, parts_b, cnt3, p, b)


def kernel(x, edge_index, Wl0, bl0, Wr0, Wl1, bl1, Wr1, Wl2, bl2, Wr2):
  srcr = edge_index[0].reshape(_ROWS, _C)
  dstr = edge_index[1].reshape(_ROWS, _C)
  w0 = jnp.concatenate([Wl0, Wr0], axis=1)
  w1 = jnp.concatenate([Wl1, Wr1], axis=1)
  w2 = jnp.concatenate([Wl2, Wr2], axis=1)
  b0 = bl0.reshape(1, -1)
  b1 = bl1.reshape(1, -1)
  b2 = bl2.reshape(1, -1)

  sp = (_D, _D, _D, _D)

  y0 = _stage0(x, w0, sp + (128,))
  r0 = _agg4c(y0[0], y0[1], y0[2], y0[3], srcr, dstr)
  parts0 = [q.reshape(_NC, _N, _D) for q in r0[:4]]
  cnt3 = r0[4].reshape(_NC, _N, 1)

  y1 = _mid(parts0, cnt3, y0[4], b0, w1, sp + (128,))
  parts1 = _agg4(y1[0], y1[1], y1[2], y1[3], srcr, dstr)
  parts1 = [q.reshape(_NC, _N, _D) for q in parts1]

  y2 = _mid(parts1, cnt3, y1[4], b1, w2, (_D, _D, 64))
  parts2 = _agg2(y2[0], y2[1], srcr, dstr)
  parts2 = [q.reshape(_NC, _N, _D) for q in parts2]

  return _final(parts2[0], parts2[1], cnt3, y2[2], b2)


# speedup vs baseline: 9.4467x; 1.6018x over previous
import functools

import jax
import jax.numpy as jnp
from jax import lax
from jax.experimental import pallas as pl
from jax.experimental.pallas import tpu as pltpu
from jax.experimental.pallas import tpu_sc as plsc

_N = 10000
_E = 320000
_D = 32
_NC = 2
_NS = 16
_NW = _NC * _NS
_EPW = 9984
_C = 624
_NCH = _EPW // _C
_NPAIR = _NCH // 2
_XC = 128
_XBASE = _EPW * _NW
_RPT = 624
_EXTRA_START = _RPT * _NS
_EXTRA = _N - _EXTRA_START


def _make_agg(nparts, with_cnt):
  mesh = plsc.VectorSubcoreMesh(core_axis_name="c", subcore_axis_name="s")
  out_type = [jax.ShapeDtypeStruct((_NC * _N, _D), jnp.float32)
              for _ in range(nparts)]
  if with_cnt:
    out_type.append(jax.ShapeDtypeStruct((_NC * _N,), jnp.float32))
  scratch = [
      pltpu.VMEM((_EPW,), jnp.int32),
      pltpu.VMEM((_EPW,), jnp.int32),
      pltpu.VMEM((_XC,), jnp.int32),
      pltpu.VMEM((_XC,), jnp.int32),
      pltpu.VMEM((_C, _D), jnp.float32),
      pltpu.VMEM((_C, _D), jnp.float32),
      pltpu.VMEM((_RPT + _EXTRA, _D), jnp.float32),
      pltpu.VMEM((_RPT + _EXTRA, _D), jnp.float32),
      pltpu.VMEM_SHARED((_N, _D), jnp.float32),
      pltpu.SemaphoreType.DMA,
      pltpu.SemaphoreType.DMA,
      pltpu.SemaphoreType.DMA,
      pltpu.SemaphoreType.DMA,
  ]
  if with_cnt:
    scratch += [
        pltpu.VMEM((_C,), jnp.float32),
        pltpu.VMEM((_RPT + _EXTRA,), jnp.float32),
        pltpu.VMEM_SHARED((_N,), jnp.float32),
        pltpu.SemaphoreType.DMA,
    ]

  def body(*refs):
    ys = refs[:nparts]
    srcr_hbm, dstr_hbm = refs[nparts:nparts + 2]
    outs = refs[nparts + 2:2 * nparts + 2]
    rest = refs[2 * nparts + 2:]
    if with_cnt:
      (cnt_hbm, sidx2, didx2, sidxe, didxe, rowsA, rowsB, zbuf, rbuf, acc,
       semGA, semGB, semSA, semSB, ones, cbuf, cacc, semC) = rest
    else:
      (sidx2, didx2, sidxe, didxe, rowsA, rowsB, zbuf, rbuf, acc,
       semGA, semGB, semSA, semSB) = rest
    cid = lax.axis_index("c")
    sid = lax.axis_index("s")
    wid = sid * _NC + cid

    pltpu.sync_copy(srcr_hbm.at[pl.ds(wid * _EPW, _EPW)], sidx2)
    pltpu.sync_copy(dstr_hbm.at[pl.ds(wid * _EPW, _EPW)], didx2)

    @pl.when(wid < 4)
    def _():
      xoff = pl.multiple_of(_XBASE + wid * _XC, 8)
      pltpu.sync_copy(srcr_hbm.at[pl.ds(xoff, _XC)], sidxe)
      pltpu.sync_copy(dstr_hbm.at[pl.ds(xoff, _XC)], didxe)

    def _zrow(i, carry):
      for k in range(_D // 16):
        zbuf[i, pl.ds(k * 16, 16)] = jnp.zeros((16,), jnp.float32)
      return carry
    lax.fori_loop(0, _RPT + _EXTRA, _zrow, 0)

    if with_cnt:
      for k in range(_C // 16):
        ones[pl.ds(k * 16, 16)] = jnp.ones((16,), jnp.float32)
      for k in range((_RPT + _EXTRA) // 16):
        cbuf[pl.ds(k * 16, 16)] = jnp.zeros((16,), jnp.float32)
      pltpu.sync_copy(cbuf.at[pl.ds(0, _RPT)],
                      cacc.at[pl.ds(sid * _RPT, _RPT)])

      @pl.when(sid == _NS - 1)
      def _():
        pltpu.sync_copy(cbuf.at[pl.ds(0, _EXTRA)],
                        cacc.at[pl.ds(_EXTRA_START, _EXTRA)])

    for h in range(nparts):
      y_hbm = ys[h]
      out_hbm = outs[h]
      do_cnt = with_cnt and h == 0

      pltpu.sync_copy(zbuf.at[pl.ds(0, _RPT)],
                      acc.at[pl.ds(sid * _RPT, _RPT)])

      @pl.when(sid == _NS - 1)
      def _():
        pltpu.sync_copy(zbuf.at[pl.ds(0, _EXTRA)],
                        acc.at[pl.ds(_EXTRA_START, _EXTRA)])

      plsc.subcore_barrier()

      pltpu.async_copy(y_hbm.at[sidx2.at[pl.ds(0, _C)]], rowsA, semGA)

      def pair(j2, carry):
        oa = pl.multiple_of(j2 * 2 * _C, 8)
        ob = pl.multiple_of((j2 * 2 + 1) * _C, 8)
        oc = pl.multiple_of((j2 * 2 + 2) * _C, 8)
        pltpu.make_async_copy(y_hbm.at[sidx2.at[pl.ds(oa, _C)]], rowsA,
                              semGA).wait()

        @pl.when(j2 > 0)
        def _():
          pltpu.make_async_copy(rowsB, acc.at[didx2.at[pl.ds(0, _C)]],
                                semSB).wait()

        pltpu.async_copy(y_hbm.at[sidx2.at[pl.ds(ob, _C)]], rowsB, semGB)
        pltpu.async_copy(rowsA, acc.at[didx2.at[pl.ds(oa, _C)]], semSA,
                         add=True)
        if do_cnt:
          pltpu.async_copy(ones, cacc.at[didx2.at[pl.ds(oa, _C)]], semC,
                           add=True)
          pltpu.async_copy(ones, cacc.at[didx2.at[pl.ds(ob, _C)]], semC,
                           add=True)
        pltpu.make_async_copy(y_hbm.at[sidx2.at[pl.ds(ob, _C)]], rowsB,
                              semGB).wait()
        pltpu.make_async_copy(rowsA, acc.at[didx2.at[pl.ds(oa, _C)]],
                              semSA).wait()
        pltpu.async_copy(rowsB, acc.at[didx2.at[pl.ds(ob, _C)]], semSB,
                         add=True)

        @pl.when(j2 < _NPAIR - 1)
        def _():
          pltpu.async_copy(y_hbm.at[sidx2.at[pl.ds(oc, _C)]], rowsA, semGA)

        return carry

      lax.fori_loop(0, _NPAIR, pair, 0)
      pltpu.make_async_copy(rowsB, acc.at[didx2.at[pl.ds(0, _C)]],
                            semSB).wait()

      @pl.when(wid < 4)
      def _():
        pltpu.async_copy(y_hbm.at[sidxe], rowsA.at[pl.ds(0, _XC)],
                         semGA).wait()
        pltpu.sync_copy(rowsA.at[pl.ds(0, _XC)], acc.at[didxe], add=True)
        if do_cnt:
          pltpu.sync_copy(ones.at[pl.ds(0, _XC)], cacc.at[didxe], add=True)

      if do_cnt:
        def drain(i, carry):
          pltpu.make_async_copy(ones, cacc.at[didx2.at[pl.ds(0, _C)]],
                                semC).wait()
          return carry
        lax.fori_loop(0, 2 * _NPAIR, drain, 0)

      plsc.subcore_barrier()

      rs = sid * _RPT
      pltpu.sync_copy(acc.at[pl.ds(rs, _RPT)], rbuf.at[pl.ds(0, _RPT)])
      pltpu.sync_copy(rbuf.at[pl.ds(0, _RPT)],
                      out_hbm.at[pl.ds(cid * _N + rs, _RPT)])
      if do_cnt:
        pltpu.sync_copy(cacc.at[pl.ds(rs, _RPT)], cbuf.at[pl.ds(0, _RPT)])
        pltpu.sync_copy(cbuf.at[pl.ds(0, _RPT)],
                        cnt_hbm.at[pl.ds(cid * _N + rs, _RPT)])

      @pl.when(sid == _NS - 1)
      def _():
        pltpu.sync_copy(acc.at[pl.ds(_EXTRA_START, _EXTRA)],
                        rbuf.at[pl.ds(_RPT, _EXTRA)])
        pltpu.sync_copy(rbuf.at[pl.ds(_RPT, _EXTRA)],
                        out_hbm.at[pl.ds(cid * _N + _EXTRA_START, _EXTRA)])
        if do_cnt:
          pltpu.sync_copy(cacc.at[pl.ds(_EXTRA_START, _EXTRA)],
                          cbuf.at[pl.ds(_RPT, _EXTRA)])
          pltpu.sync_copy(cbuf.at[pl.ds(_RPT, _EXTRA)],
                          cnt_hbm.at[pl.ds(cid * _N + _EXTRA_START, _EXTRA)])

      if h + 1 < nparts:
        plsc.subcore_barrier()

  return pl.kernel(
      body, out_type=out_type, mesh=mesh, scratch_types=scratch,
      compiler_params=pltpu.CompilerParams(use_tc_tiling_on_sc=False))


_agg4c = _make_agg(4, True)
_agg4 = _make_agg(4, False)
_agg2 = _make_agg(2, False)


_R = 1000
_G = _N // _R


def _s0_body(x_ref, w_ref, *out_refs, splits):
  res = jnp.dot(x_ref[...], w_ref[...], preferred_element_type=jnp.float32)
  off = 0
  for r, s in zip(out_refs, splits):
    r[...] = res[:, off:off + s]
    off += s


def _stage0(x, wcat, splits):
  K = x.shape[1]
  return pl.pallas_call(
      functools.partial(_s0_body, splits=splits),
      grid=(_G,),
      in_specs=[pl.BlockSpec((_R, K), lambda i: (i, 0)),
                pl.BlockSpec((K, sum(splits)), lambda i: (0, 0))],
      out_specs=[pl.BlockSpec((_R, s), lambda i: (i, 0)) for s in splits],
      out_shape=[jax.ShapeDtypeStruct((_N, s), jnp.float32) for s in splits],
  )(x, wcat)


def _mid_body(*refs, nq, splits):
  part_refs = refs[:nq]
  cnt_ref, p_ref, b_ref, w_ref = refs[nq:nq + 4]
  out_refs = refs[nq + 4:]
  cnt = cnt_ref[0] + cnt_ref[1]
  inv = 1.0 / jnp.maximum(cnt, 1.0)
  mean = jnp.concatenate([(pr[0] + pr[1]) * inv for pr in part_refs], axis=1)
  h = jnp.maximum(mean + b_ref[...] + p_ref[...], 0.0)
  res = jnp.dot(h, w_ref[...], preferred_element_type=jnp.float32)
  off = 0
  for r, s in zip(out_refs, splits):
    r[...] = res[:, off:off + s]
    off += s


def _mid(parts_list, cnt3, p, b, wcat, splits):
  nq = len(parts_list)
  Din = nq * _D
  return pl.pallas_call(
      functools.partial(_mid_body, nq=nq, splits=splits),
      grid=(_G,),
      in_specs=([pl.BlockSpec((_NC, _R, _D), lambda i: (0, i, 0))] * nq
                + [pl.BlockSpec((_NC, _R, 1), lambda i: (0, i, 0)),
                   pl.BlockSpec((_R, Din), lambda i: (i, 0)),
                   pl.BlockSpec((1, Din), lambda i: (0, 0)),
                   pl.BlockSpec((Din, sum(splits)), lambda i: (0, 0))]),
      out_specs=[pl.BlockSpec((_R, s), lambda i: (i, 0)) for s in splits],
      out_shape=[jax.ShapeDtypeStruct((_N, s), jnp.float32) for s in splits],
  )(*parts_list, cnt3, p, b, wcat)


def _final_body(pa_ref, pb_ref, cnt_ref, p_ref, b_ref, o_ref):
  cnt = cnt_ref[0] + cnt_ref[1]
  inv = 1.0 / jnp.maximum(cnt, 1.0)
  mean = jnp.concatenate([(pa_ref[0] + pa_ref[1]) * inv,
                          (pb_ref[0] + pb_ref[1]) * inv], axis=1)
  o = mean + b_ref[...] + p_ref[...]
  m = jnp.max(o, axis=1, keepdims=True)
  lse = jnp.log(jnp.sum(jnp.exp(o - m), axis=1, keepdims=True))
  o_ref[...] = o - m - lse


def _final(parts_a, parts_b, cnt3, p, b):
  D = 2 * _D
  return pl.pallas_call(
      _final_body,
      grid=(_G,),
      in_specs=[pl.BlockSpec((_NC, _R, _D), lambda i: (0, i, 0)),
                pl.BlockSpec((_NC, _R, _D), lambda i: (0, i, 0)),
                pl.BlockSpec((_NC, _R, 1), lambda i: (0, i, 0)),
                pl.BlockSpec((_R, D), lambda i: (i, 0)),
                pl.BlockSpec((1, D), lambda i: (0, 0))],
      out_specs=pl.BlockSpec((_R, D), lambda i: (i, 0)),
      out_shape=jax.ShapeDtypeStruct((_N, D), jnp.float32),
  )(parts_a, parts_b, cnt3, p, b)


def kernel(x, edge_index, Wl0, bl0, Wr0, Wl1, bl1, Wr1, Wl2, bl2, Wr2):
  srcr = edge_index[0]
  dstr = edge_index[1]
  w0 = jnp.concatenate([Wl0, Wr0], axis=1)
  w1 = jnp.concatenate([Wl1, Wr1], axis=1)
  w2 = jnp.concatenate([Wl2, Wr2], axis=1)
  b0 = bl0.reshape(1, -1)
  b1 = bl1.reshape(1, -1)
  b2 = bl2.reshape(1, -1)

  sp = (_D, _D, _D, _D)

  y0 = _stage0(x, w0, sp + (128,))
  r0 = _agg4c(y0[0], y0[1], y0[2], y0[3], srcr, dstr)
  parts0 = [q.reshape(_NC, _N, _D) for q in r0[:4]]
  cnt3 = r0[4].reshape(_NC, _N, 1)

  y1 = _mid(parts0, cnt3, y0[4], b0, w1, sp + (128,))
  parts1 = _agg4(y1[0], y1[1], y1[2], y1[3], srcr, dstr)
  parts1 = [q.reshape(_NC, _N, _D) for q in parts1]

  y2 = _mid(parts1, cnt3, y1[4], b1, w2, (_D, _D, 64))
  parts2 = _agg2(y2[0], y2[1], srcr, dstr)
  parts2 = [q.reshape(_NC, _N, _D) for q in parts2]

  return _final(parts2[0], parts2[1], cnt3, y2[2], b2)


# speedup vs baseline: 9.8961x; 1.0476x over previous
import functools

import jax
import jax.numpy as jnp
from jax import lax
from jax.experimental import pallas as pl
from jax.experimental.pallas import tpu as pltpu
from jax.experimental.pallas import tpu_sc as plsc

_N = 10000
_E = 320000
_D = 32
_NC = 2
_NS = 16
_NW = _NC * _NS
_EPW = 9984
_C = 416
_NCH = _EPW // _C
_NBUF = 4
_NR = _NCH // _NBUF
_XC = 128
_XBASE = _EPW * _NW
_RPT = 624
_EXTRA_START = _RPT * _NS
_EXTRA = _N - _EXTRA_START


def _make_agg(nparts, with_cnt):
  mesh = plsc.VectorSubcoreMesh(core_axis_name="c", subcore_axis_name="s")
  out_type = [jax.ShapeDtypeStruct((_NC * _N, _D), jnp.float32)
              for _ in range(nparts)]
  if with_cnt:
    out_type.append(jax.ShapeDtypeStruct((_NC * _N,), jnp.float32))
  scratch = [
      pltpu.VMEM((_EPW,), jnp.int32),
      pltpu.VMEM((_EPW,), jnp.int32),
      pltpu.VMEM((_XC,), jnp.int32),
      pltpu.VMEM((_XC,), jnp.int32),
      *([pltpu.VMEM((_C, _D), jnp.float32)] * _NBUF),
      pltpu.VMEM((_RPT // 3, _D), jnp.float32),
      pltpu.VMEM((_RPT + _EXTRA, _D), jnp.float32),
      pltpu.VMEM_SHARED((_N, _D), jnp.float32),
      *([pltpu.SemaphoreType.DMA] * _NBUF),
      *([pltpu.SemaphoreType.DMA] * _NBUF),
  ]
  if with_cnt:
    scratch += [
        pltpu.VMEM((_C,), jnp.float32),
        pltpu.VMEM((_RPT + _EXTRA,), jnp.float32),
        pltpu.VMEM_SHARED((_N,), jnp.float32),
        pltpu.SemaphoreType.DMA,
    ]

  def body(*refs):
    ys = refs[:nparts]
    srcr_hbm, dstr_hbm = refs[nparts:nparts + 2]
    outs = refs[nparts + 2:2 * nparts + 2]
    rest = refs[2 * nparts + 2:]
    if with_cnt:
      cnt_hbm = rest[0]
      rest = rest[1:]
    (sidx2, didx2, sidxe, didxe) = rest[:4]
    rows = rest[4:4 + _NBUF]
    zbuf, rbuf, acc = rest[4 + _NBUF:7 + _NBUF]
    semG = rest[7 + _NBUF:7 + 2 * _NBUF]
    semS = rest[7 + 2 * _NBUF:7 + 3 * _NBUF]
    if with_cnt:
      ones, cbuf, cacc, semC = rest[7 + 3 * _NBUF:]
    cid = lax.axis_index("c")
    sid = lax.axis_index("s")
    wid = sid * _NC + cid

    pltpu.sync_copy(srcr_hbm.at[pl.ds(wid * _EPW, _EPW)], sidx2)
    pltpu.sync_copy(dstr_hbm.at[pl.ds(wid * _EPW, _EPW)], didx2)

    @pl.when(wid < 4)
    def _():
      xoff = pl.multiple_of(_XBASE + wid * _XC, 8)
      pltpu.sync_copy(srcr_hbm.at[pl.ds(xoff, _XC)], sidxe)
      pltpu.sync_copy(dstr_hbm.at[pl.ds(xoff, _XC)], didxe)

    def _zrow(i, carry):
      for k in range(_D // 16):
        zbuf[i, pl.ds(k * 16, 16)] = jnp.zeros((16,), jnp.float32)
      return carry
    lax.fori_loop(0, _RPT // 3, _zrow, 0)

    if with_cnt:
      for k in range(_C // 16):
        ones[pl.ds(k * 16, 16)] = jnp.ones((16,), jnp.float32)
      for k in range((_RPT + _EXTRA) // 16):
        cbuf[pl.ds(k * 16, 16)] = jnp.zeros((16,), jnp.float32)
      pltpu.sync_copy(cbuf.at[pl.ds(0, _RPT)],
                      cacc.at[pl.ds(sid * _RPT, _RPT)])

      @pl.when(sid == _NS - 1)
      def _():
        pltpu.sync_copy(cbuf.at[pl.ds(0, _EXTRA)],
                        cacc.at[pl.ds(_EXTRA_START, _EXTRA)])

    for h in range(nparts):
      y_hbm = ys[h]
      out_hbm = outs[h]
      do_cnt = with_cnt and h == 0

      for z in range(3):
        pltpu.sync_copy(zbuf,
                        acc.at[pl.ds(sid * _RPT + z * (_RPT // 3),
                                     _RPT // 3)])

      @pl.when(sid == _NS - 1)
      def _():
        pltpu.sync_copy(zbuf.at[pl.ds(0, _EXTRA)],
                        acc.at[pl.ds(_EXTRA_START, _EXTRA)])

      plsc.subcore_barrier()

      def ring(j4, carry):
        for k in range(_NBUF):
          off = pl.multiple_of((j4 * _NBUF + k) * _C, 8)

          @pl.when(j4 > 0)
          def _():
            pltpu.make_async_copy(rows[k], acc.at[didx2.at[pl.ds(0, _C)]],
                                  semS[k]).wait()

          pltpu.async_copy(y_hbm.at[sidx2.at[pl.ds(off, _C)]], rows[k],
                           semG[k])
          if do_cnt:
            pltpu.async_copy(ones, cacc.at[didx2.at[pl.ds(off, _C)]], semC,
                             add=True)
        for k in range(_NBUF):
          off = pl.multiple_of((j4 * _NBUF + k) * _C, 8)
          pltpu.make_async_copy(y_hbm.at[sidx2.at[pl.ds(off, _C)]], rows[k],
                                semG[k]).wait()
          pltpu.async_copy(rows[k], acc.at[didx2.at[pl.ds(off, _C)]],
                           semS[k], add=True)
        return carry

      lax.fori_loop(0, _NR, ring, 0)
      for k in range(_NBUF):
        pltpu.make_async_copy(rows[k], acc.at[didx2.at[pl.ds(0, _C)]],
                              semS[k]).wait()

      @pl.when(wid < 4)
      def _():
        pltpu.async_copy(y_hbm.at[sidxe], rows[0].at[pl.ds(0, _XC)],
                         semG[0]).wait()
        pltpu.sync_copy(rows[0].at[pl.ds(0, _XC)], acc.at[didxe], add=True)
        if do_cnt:
          pltpu.sync_copy(ones.at[pl.ds(0, _XC)], cacc.at[didxe], add=True)

      if do_cnt:
        def drain(i, carry):
          pltpu.make_async_copy(ones, cacc.at[didx2.at[pl.ds(0, _C)]],
                                semC).wait()
          return carry
        lax.fori_loop(0, _NCH, drain, 0)

      plsc.subcore_barrier()

      rs = sid * _RPT
      pltpu.sync_copy(acc.at[pl.ds(rs, _RPT)], rbuf.at[pl.ds(0, _RPT)])
      pltpu.sync_copy(rbuf.at[pl.ds(0, _RPT)],
                      out_hbm.at[pl.ds(cid * _N + rs, _RPT)])
      if do_cnt:
        pltpu.sync_copy(cacc.at[pl.ds(rs, _RPT)], cbuf.at[pl.ds(0, _RPT)])
        pltpu.sync_copy(cbuf.at[pl.ds(0, _RPT)],
                        cnt_hbm.at[pl.ds(cid * _N + rs, _RPT)])

      @pl.when(sid == _NS - 1)
      def _():
        pltpu.sync_copy(acc.at[pl.ds(_EXTRA_START, _EXTRA)],
                        rbuf.at[pl.ds(_RPT, _EXTRA)])
        pltpu.sync_copy(rbuf.at[pl.ds(_RPT, _EXTRA)],
                        out_hbm.at[pl.ds(cid * _N + _EXTRA_START, _EXTRA)])
        if do_cnt:
          pltpu.sync_copy(cacc.at[pl.ds(_EXTRA_START, _EXTRA)],
                          cbuf.at[pl.ds(_RPT, _EXTRA)])
          pltpu.sync_copy(cbuf.at[pl.ds(_RPT, _EXTRA)],
                          cnt_hbm.at[pl.ds(cid * _N + _EXTRA_START, _EXTRA)])

      if h + 1 < nparts:
        plsc.subcore_barrier()

  return pl.kernel(
      body, out_type=out_type, mesh=mesh, scratch_types=scratch,
      compiler_params=pltpu.CompilerParams(use_tc_tiling_on_sc=False))


_agg4c = _make_agg(4, True)
_agg4 = _make_agg(4, False)
_agg2 = _make_agg(2, False)


_R = 1000
_G = _N // _R


def _s0_body(x_ref, w_ref, *out_refs, splits):
  res = jnp.dot(x_ref[...], w_ref[...], preferred_element_type=jnp.float32)
  off = 0
  for r, s in zip(out_refs, splits):
    r[...] = res[:, off:off + s]
    off += s


def _stage0(x, wcat, splits):
  K = x.shape[1]
  return pl.pallas_call(
      functools.partial(_s0_body, splits=splits),
      grid=(_G,),
      in_specs=[pl.BlockSpec((_R, K), lambda i: (i, 0)),
                pl.BlockSpec((K, sum(splits)), lambda i: (0, 0))],
      out_specs=[pl.BlockSpec((_R, s), lambda i: (i, 0)) for s in splits],
      out_shape=[jax.ShapeDtypeStruct((_N, s), jnp.float32) for s in splits],
  )(x, wcat)


def _mid_body(*refs, nq, splits):
  part_refs = refs[:nq]
  cnt_ref, p_ref, b_ref, w_ref = refs[nq:nq + 4]
  out_refs = refs[nq + 4:]
  cnt = cnt_ref[0] + cnt_ref[1]
  inv = 1.0 / jnp.maximum(cnt, 1.0)
  mean = jnp.concatenate([(pr[0] + pr[1]) * inv for pr in part_refs], axis=1)
  h = jnp.maximum(mean + b_ref[...] + p_ref[...], 0.0)
  res = jnp.dot(h, w_ref[...], preferred_element_type=jnp.float32)
  off = 0
  for r, s in zip(out_refs, splits):
    r[...] = res[:, off:off + s]
    off += s


def _mid(parts_list, cnt3, p, b, wcat, splits):
  nq = len(parts_list)
  Din = nq * _D
  return pl.pallas_call(
      functools.partial(_mid_body, nq=nq, splits=splits),
      grid=(_G,),
      in_specs=([pl.BlockSpec((_NC, _R, _D), lambda i: (0, i, 0))] * nq
                + [pl.BlockSpec((_NC, _R, 1), lambda i: (0, i, 0)),
                   pl.BlockSpec((_R, Din), lambda i: (i, 0)),
                   pl.BlockSpec((1, Din), lambda i: (0, 0)),
                   pl.BlockSpec((Din, sum(splits)), lambda i: (0, 0))]),
      out_specs=[pl.BlockSpec((_R, s), lambda i: (i, 0)) for s in splits],
      out_shape=[jax.ShapeDtypeStruct((_N, s), jnp.float32) for s in splits],
  )(*parts_list, cnt3, p, b, wcat)


def _final_body(pa_ref, pb_ref, cnt_ref, p_ref, b_ref, o_ref):
  cnt = cnt_ref[0] + cnt_ref[1]
  inv = 1.0 / jnp.maximum(cnt, 1.0)
  mean = jnp.concatenate([(pa_ref[0] + pa_ref[1]) * inv,
                          (pb_ref[0] + pb_ref[1]) * inv], axis=1)
  o = mean + b_ref[...] + p_ref[...]
  m = jnp.max(o, axis=1, keepdims=True)
  lse = jnp.log(jnp.sum(jnp.exp(o - m), axis=1, keepdims=True))
  o_ref[...] = o - m - lse


def _final(parts_a, parts_b, cnt3, p, b):
  D = 2 * _D
  return pl.pallas_call(
      _final_body,
      grid=(_G,),
      in_specs=[pl.BlockSpec((_NC, _R, _D), lambda i: (0, i, 0)),
                pl.BlockSpec((_NC, _R, _D), lambda i: (0, i, 0)),
                pl.BlockSpec((_NC, _R, 1), lambda i: (0, i, 0)),
                pl.BlockSpec((_R, D), lambda i: (i, 0)),
                pl.BlockSpec((1, D), lambda i: (0, 0))],
      out_specs=pl.BlockSpec((_R, D), lambda i: (i, 0)),
      out_shape=jax.ShapeDtypeStruct((_N, D), jnp.float32),
  )(parts_a, parts_b, cnt3, p, b)


def kernel(x, edge_index, Wl0, bl0, Wr0, Wl1, bl1, Wr1, Wl2, bl2, Wr2):
  srcr = edge_index[0]
  dstr = edge_index[1]
  w0 = jnp.concatenate([Wl0, Wr0], axis=1)
  w1 = jnp.concatenate([Wl1, Wr1], axis=1)
  w2 = jnp.concatenate([Wl2, Wr2], axis=1)
  b0 = bl0.reshape(1, -1)
  b1 = bl1.reshape(1, -1)
  b2 = bl2.reshape(1, -1)

  sp = (_D, _D, _D, _D)

  y0 = _stage0(x, w0, sp + (128,))
  r0 = _agg4c(y0[0], y0[1], y0[2], y0[3], srcr, dstr)
  parts0 = [q.reshape(_NC, _N, _D) for q in r0[:4]]
  cnt3 = r0[4].reshape(_NC, _N, 1)

  y1 = _mid(parts0, cnt3, y0[4], b0, w1, sp + (128,))
  parts1 = _agg4(y1[0], y1[1], y1[2], y1[3], srcr, dstr)
  parts1 = [q.reshape(_NC, _N, _D) for q in parts1]

  y2 = _mid(parts1, cnt3, y1[4], b1, w2, (_D, _D, 64))
  parts2 = _agg2(y2[0], y2[1], srcr, dstr)
  parts2 = [q.reshape(_NC, _N, _D) for q in parts2]

  return _final(parts2[0], parts2[1], cnt3, y2[2], b2)


# speedup vs baseline: 11.6797x; 1.1802x over previous
import functools

import jax
import jax.numpy as jnp
from jax import lax
from jax.experimental import pallas as pl
from jax.experimental.pallas import tpu as pltpu
from jax.experimental.pallas import tpu_sc as plsc

_N = 10000
_E = 320000
_D = 32
_NC = 2
_NS = 16
_NW = _NC * _NS
_EPW = 9984
_C = 416
_NCH = _EPW // _C
_NBUF = 4
_NR = _NCH // _NBUF
_XC = 128
_XBASE = _EPW * _NW
_RPT = 624
_ZR = _RPT // 3
_EXTRA_START = _RPT * _NS
_EXTRA = _N - _EXTRA_START


def _make_agg(nparts, with_cnt):
  mesh = plsc.VectorSubcoreMesh(core_axis_name="c", subcore_axis_name="s")
  shift = nparts.bit_length() - 1
  out_type = [jax.ShapeDtypeStruct((_NC * _N, _D * nparts), jnp.float32)]
  if with_cnt:
    out_type.append(jax.ShapeDtypeStruct((_NC * _N,), jnp.float32))
  scratch = [
      pltpu.VMEM((_EPW,), jnp.int32),
      pltpu.VMEM((_EPW,), jnp.int32),
      pltpu.VMEM((_EPW,), jnp.int32),
      pltpu.VMEM((_XC,), jnp.int32),
      pltpu.VMEM((_XC,), jnp.int32),
      pltpu.VMEM((_XC,), jnp.int32),
      *([pltpu.VMEM((_C, _D), jnp.float32)] * _NBUF),
      pltpu.VMEM((_ZR, _D), jnp.float32),
      pltpu.VMEM_SHARED((_N, _D), jnp.float32),
      *([pltpu.SemaphoreType.DMA] * _NBUF),
      *([pltpu.SemaphoreType.DMA] * _NBUF),
  ]
  if with_cnt:
    scratch += [
        pltpu.VMEM((_C,), jnp.float32),
        pltpu.VMEM((_RPT + _EXTRA,), jnp.float32),
        pltpu.VMEM_SHARED((_N,), jnp.float32),
        pltpu.SemaphoreType.DMA,
    ]

  def body(y_hbm, srcr_hbm, dstr_hbm, out_hbm, *rest):
    if with_cnt:
      cnt_hbm = rest[0]
      rest = rest[1:]
    (sidx2, didx2, sidxh, sidxe, didxe, sidxeh) = rest[:6]
    rows = rest[6:6 + _NBUF]
    zbuf, acc = rest[6 + _NBUF:8 + _NBUF]
    semG = rest[8 + _NBUF:8 + 2 * _NBUF]
    semS = rest[8 + 2 * _NBUF:8 + 3 * _NBUF]
    if with_cnt:
      ones, cbuf, cacc, semC = rest[8 + 3 * _NBUF:]
    cid = lax.axis_index("c")
    sid = lax.axis_index("s")
    wid = sid * _NC + cid

    pltpu.sync_copy(srcr_hbm.at[pl.ds(wid * _EPW, _EPW)], sidx2)
    pltpu.sync_copy(dstr_hbm.at[pl.ds(wid * _EPW, _EPW)], didx2)

    @pl.when(wid < 4)
    def _():
      xoff = pl.multiple_of(_XBASE + wid * _XC, 8)
      pltpu.sync_copy(srcr_hbm.at[pl.ds(xoff, _XC)], sidxe)
      pltpu.sync_copy(dstr_hbm.at[pl.ds(xoff, _XC)], didxe)

    def _zrow(i, carry):
      for k in range(_D // 16):
        zbuf[i, pl.ds(k * 16, 16)] = jnp.zeros((16,), jnp.float32)
      return carry
    lax.fori_loop(0, _ZR, _zrow, 0)

    if with_cnt:
      for k in range(_C // 16):
        ones[pl.ds(k * 16, 16)] = jnp.ones((16,), jnp.float32)
      for k in range((_RPT + _EXTRA) // 16):
        cbuf[pl.ds(k * 16, 16)] = jnp.zeros((16,), jnp.float32)
      pltpu.sync_copy(cbuf.at[pl.ds(0, _RPT)],
                      cacc.at[pl.ds(sid * _RPT, _RPT)])

      @pl.when(sid == _NS - 1)
      def _():
        pltpu.sync_copy(cbuf.at[pl.ds(0, _EXTRA)],
                        cacc.at[pl.ds(_EXTRA_START, _EXTRA)])

    for h in range(nparts):
      do_cnt = with_cnt and h == 0
      hvec = jnp.full((16,), h, jnp.int32)

      def _sh(i, carry):
        v = sidx2[pl.ds(i * 16, 16)]
        sidxh[pl.ds(i * 16, 16)] = (v << shift) + hvec
        return carry
      lax.fori_loop(0, _EPW // 16, _sh, 0)

      @pl.when(wid < 4)
      def _():
        def _she(i, carry):
          v = sidxe[pl.ds(i * 16, 16)]
          sidxeh[pl.ds(i * 16, 16)] = (v << shift) + hvec
          return carry
        lax.fori_loop(0, _XC // 16, _she, 0)

      for z in range(3):
        pltpu.sync_copy(zbuf, acc.at[pl.ds(sid * _RPT + z * _ZR, _ZR)])

      @pl.when(sid == _NS - 1)
      def _():
        pltpu.sync_copy(zbuf.at[pl.ds(0, _EXTRA)],
                        acc.at[pl.ds(_EXTRA_START, _EXTRA)])

      plsc.subcore_barrier()

      def ring(j4, carry):
        for k in range(_NBUF):
          off = pl.multiple_of((j4 * _NBUF + k) * _C, 8)

          @pl.when(j4 > 0)
          def _():
            pltpu.make_async_copy(rows[k], acc.at[didx2.at[pl.ds(0, _C)]],
                                  semS[k]).wait()

          pltpu.async_copy(y_hbm.at[sidxh.at[pl.ds(off, _C)]], rows[k],
                           semG[k])
          if do_cnt:
            pltpu.async_copy(ones, cacc.at[didx2.at[pl.ds(off, _C)]], semC,
                             add=True)
        for k in range(_NBUF):
          off = pl.multiple_of((j4 * _NBUF + k) * _C, 8)
          pltpu.make_async_copy(y_hbm.at[sidxh.at[pl.ds(off, _C)]], rows[k],
                                semG[k]).wait()
          pltpu.async_copy(rows[k], acc.at[didx2.at[pl.ds(off, _C)]],
                           semS[k], add=True)
        return carry

      lax.fori_loop(0, _NR, ring, 0)
      for k in range(_NBUF):
        pltpu.make_async_copy(rows[k], acc.at[didx2.at[pl.ds(0, _C)]],
                              semS[k]).wait()

      @pl.when(wid < 4)
      def _():
        pltpu.async_copy(y_hbm.at[sidxeh], rows[0].at[pl.ds(0, _XC)],
                         semG[0]).wait()
        pltpu.sync_copy(rows[0].at[pl.ds(0, _XC)], acc.at[didxe], add=True)
        if do_cnt:
          pltpu.sync_copy(ones.at[pl.ds(0, _XC)], cacc.at[didxe], add=True)

      if do_cnt:
        def drain(i, carry):
          pltpu.make_async_copy(ones, cacc.at[didx2.at[pl.ds(0, _C)]],
                                semC).wait()
          return carry
        lax.fori_loop(0, _NCH, drain, 0)

      plsc.subcore_barrier()

      rs = sid * _RPT
      pltpu.sync_copy(acc.at[pl.ds(rs, _RPT)],
                      out_hbm.at[pl.ds(cid * _N + rs, _RPT),
                                 pl.ds(h * _D, _D)])
      if do_cnt:
        pltpu.sync_copy(cacc.at[pl.ds(rs, _RPT)],
                        cnt_hbm.at[pl.ds(cid * _N + rs, _RPT)])

      @pl.when(sid == _NS - 1)
      def _():
        pltpu.sync_copy(acc.at[pl.ds(_EXTRA_START, _EXTRA)],
                        out_hbm.at[pl.ds(cid * _N + _EXTRA_START, _EXTRA),
                                   pl.ds(h * _D, _D)])
        if do_cnt:
          pltpu.sync_copy(cacc.at[pl.ds(_EXTRA_START, _EXTRA)],
                          cnt_hbm.at[pl.ds(cid * _N + _EXTRA_START, _EXTRA)])

      if h + 1 < nparts:
        plsc.subcore_barrier()

  return pl.kernel(
      body, out_type=out_type, mesh=mesh, scratch_types=scratch,
      compiler_params=pltpu.CompilerParams(use_tc_tiling_on_sc=False))


_agg4c = _make_agg(4, True)
_agg4 = _make_agg(4, False)
_agg2 = _make_agg(2, False)


_R = 1000
_G = _N // _R


def _s0_body(x_ref, w_ref, *out_refs, splits):
  res = jnp.dot(x_ref[...], w_ref[...], preferred_element_type=jnp.float32)
  off = 0
  for r, s in zip(out_refs, splits):
    r[...] = res[:, off:off + s]
    off += s


def _stage0(x, wcat, splits):
  K = x.shape[1]
  return pl.pallas_call(
      functools.partial(_s0_body, splits=splits),
      grid=(_G,),
      in_specs=[pl.BlockSpec((_R, K), lambda i: (i, 0)),
                pl.BlockSpec((K, sum(splits)), lambda i: (0, 0))],
      out_specs=[pl.BlockSpec((_R, s), lambda i: (i, 0)) for s in splits],
      out_shape=[jax.ShapeDtypeStruct((_N, s), jnp.float32) for s in splits],
  )(x, wcat)


def _mid_body(parts_ref, cnt_ref, p_ref, b_ref, w_ref, *out_refs, splits):
  cnt = cnt_ref[0] + cnt_ref[1]
  inv = 1.0 / jnp.maximum(cnt, 1.0)
  mean = (parts_ref[0] + parts_ref[1]) * inv
  h = jnp.maximum(mean + b_ref[...] + p_ref[...], 0.0)
  res = jnp.dot(h, w_ref[...], preferred_element_type=jnp.float32)
  off = 0
  for r, s in zip(out_refs, splits):
    r[...] = res[:, off:off + s]
    off += s


def _mid(parts, cnt3, p, b, wcat, splits):
  Din = parts.shape[-1]
  return pl.pallas_call(
      functools.partial(_mid_body, splits=splits),
      grid=(_G,),
      in_specs=[pl.BlockSpec((_NC, _R, Din), lambda i: (0, i, 0)),
                pl.BlockSpec((_NC, _R, 1), lambda i: (0, i, 0)),
                pl.BlockSpec((_R, Din), lambda i: (i, 0)),
                pl.BlockSpec((1, Din), lambda i: (0, 0)),
                pl.BlockSpec((Din, sum(splits)), lambda i: (0, 0))],
      out_specs=[pl.BlockSpec((_R, s), lambda i: (i, 0)) for s in splits],
      out_shape=[jax.ShapeDtypeStruct((_N, s), jnp.float32) for s in splits],
  )(parts, cnt3, p, b, wcat)


def _final_body(parts_ref, cnt_ref, p_ref, b_ref, o_ref):
  cnt = cnt_ref[0] + cnt_ref[1]
  inv = 1.0 / jnp.maximum(cnt, 1.0)
  o = (parts_ref[0] + parts_ref[1]) * inv + b_ref[...] + p_ref[...]
  m = jnp.max(o, axis=1, keepdims=True)
  lse = jnp.log(jnp.sum(jnp.exp(o - m), axis=1, keepdims=True))
  o_ref[...] = o - m - lse


def _final(parts, cnt3, p, b):
  D = parts.shape[-1]
  return pl.pallas_call(
      _final_body,
      grid=(_G,),
      in_specs=[pl.BlockSpec((_NC, _R, D), lambda i: (0, i, 0)),
                pl.BlockSpec((_NC, _R, 1), lambda i: (0, i, 0)),
                pl.BlockSpec((_R, D), lambda i: (i, 0)),
                pl.BlockSpec((1, D), lambda i: (0, 0))],
      out_specs=pl.BlockSpec((_R, D), lambda i: (i, 0)),
      out_shape=jax.ShapeDtypeStruct((_N, D), jnp.float32),
  )(parts, cnt3, p, b)


def kernel(x, edge_index, Wl0, bl0, Wr0, Wl1, bl1, Wr1, Wl2, bl2, Wr2):
  srcr = edge_index[0]
  dstr = edge_index[1]
  w0 = jnp.concatenate([Wl0, Wr0], axis=1)
  w1 = jnp.concatenate([Wl1, Wr1], axis=1)
  w2 = jnp.concatenate([Wl2, Wr2], axis=1)
  b0 = bl0.reshape(1, -1)
  b1 = bl1.reshape(1, -1)
  b2 = bl2.reshape(1, -1)

  y0, p0 = _stage0(x, w0, (128, 128))
  r0 = _agg4c(y0.reshape(4 * _N, _D), srcr, dstr)
  parts0 = r0[0].reshape(_NC, _N, 128)
  cnt3 = r0[1].reshape(_NC, _N, 1)

  y1, p1 = _mid(parts0, cnt3, p0, b0, w1, (128, 128))
  parts1 = _agg4(y1.reshape(4 * _N, _D), srcr, dstr)[0].reshape(_NC, _N, 128)

  y2, p2 = _mid(parts1, cnt3, p1, b1, w2, (64, 64))
  parts2 = _agg2(y2.reshape(2 * _N, _D), srcr, dstr)[0].reshape(_NC, _N, 64)

  return _final(parts2, cnt3, p2, b2)


# speedup vs baseline: 12.0498x; 1.0317x over previous
import functools

import jax
import jax.numpy as jnp
from jax import lax
from jax.experimental import pallas as pl
from jax.experimental.pallas import tpu as pltpu
from jax.experimental.pallas import tpu_sc as plsc

_N = 10000
_E = 320000
_D = 32
_NC = 2
_NS = 16
_NW = _NC * _NS
_EPW = 9984
_C = 416
_NCH = _EPW // _C
_NBUF = 4
_NR = _NCH // _NBUF
_XC = 128
_XBASE = _EPW * _NW
_RPT = 624
_ZR = _RPT // 3
_EXTRA_START = _RPT * _NS
_EXTRA = _N - _EXTRA_START


def _make_agg(nparts, with_cnt):
  mesh = plsc.VectorSubcoreMesh(core_axis_name="c", subcore_axis_name="s")
  shift = nparts.bit_length() - 1
  out_type = [jax.ShapeDtypeStruct((_NC * _N, _D * nparts), jnp.float32)]
  if with_cnt:
    out_type.append(jax.ShapeDtypeStruct((_NC * _N,), jnp.float32))
  scratch = [
      pltpu.VMEM((_EPW,), jnp.int32),
      pltpu.VMEM((_EPW,), jnp.int32),
      pltpu.VMEM((_XC,), jnp.int32),
      pltpu.VMEM((_XC,), jnp.int32),
      *([pltpu.VMEM((_C, _D), jnp.float32)] * _NBUF),
      pltpu.VMEM((_ZR, _D), jnp.float32),
      pltpu.VMEM_SHARED((_N, _D), jnp.float32),
      *([pltpu.SemaphoreType.DMA] * _NBUF),
      *([pltpu.SemaphoreType.DMA] * _NBUF),
  ]
  if with_cnt:
    scratch += [
        pltpu.VMEM((_C,), jnp.float32),
        pltpu.VMEM((_RPT + _EXTRA,), jnp.float32),
        pltpu.VMEM_SHARED((_N,), jnp.float32),
        pltpu.SemaphoreType.DMA,
    ]

  def body(y_hbm, srcr_hbm, dstr_hbm, out_hbm, *rest):
    if with_cnt:
      cnt_hbm = rest[0]
      rest = rest[1:]
    (sidx2, didx2, sidxe, didxe) = rest[:4]
    rows = rest[4:4 + _NBUF]
    zbuf, acc = rest[4 + _NBUF:6 + _NBUF]
    semG = rest[6 + _NBUF:6 + 2 * _NBUF]
    semS = rest[6 + 2 * _NBUF:6 + 3 * _NBUF]
    if with_cnt:
      ones, cbuf, cacc, semC = rest[6 + 3 * _NBUF:]
    cid = lax.axis_index("c")
    sid = lax.axis_index("s")
    wid = sid * _NC + cid

    pltpu.sync_copy(srcr_hbm.at[pl.ds(wid * _EPW, _EPW)], sidx2)
    pltpu.sync_copy(dstr_hbm.at[pl.ds(wid * _EPW, _EPW)], didx2)

    def _scale(i, carry):
      sidx2[pl.ds(i * 16, 16)] = sidx2[pl.ds(i * 16, 16)] << shift
      return carry
    lax.fori_loop(0, _EPW // 16, _scale, 0)

    @pl.when(wid < 4)
    def _():
      xoff = pl.multiple_of(_XBASE + wid * _XC, 8)
      pltpu.sync_copy(srcr_hbm.at[pl.ds(xoff, _XC)], sidxe)
      pltpu.sync_copy(dstr_hbm.at[pl.ds(xoff, _XC)], didxe)

      def _scalee(i, carry):
        sidxe[pl.ds(i * 16, 16)] = sidxe[pl.ds(i * 16, 16)] << shift
        return carry
      lax.fori_loop(0, _XC // 16, _scalee, 0)

    def _zrow(i, carry):
      for k in range(_D // 16):
        zbuf[i, pl.ds(k * 16, 16)] = jnp.zeros((16,), jnp.float32)
      return carry
    lax.fori_loop(0, _ZR, _zrow, 0)

    if with_cnt:
      for k in range(_C // 16):
        ones[pl.ds(k * 16, 16)] = jnp.ones((16,), jnp.float32)
      for k in range((_RPT + _EXTRA) // 16):
        cbuf[pl.ds(k * 16, 16)] = jnp.zeros((16,), jnp.float32)
      pltpu.sync_copy(cbuf.at[pl.ds(0, _RPT)],
                      cacc.at[pl.ds(sid * _RPT, _RPT)])

      @pl.when(sid == _NS - 1)
      def _():
        pltpu.sync_copy(cbuf.at[pl.ds(0, _EXTRA)],
                        cacc.at[pl.ds(_EXTRA_START, _EXTRA)])

    vrows = nparts * _N - (nparts - 1)
    for h in range(nparts):
      do_cnt = with_cnt and h == 0
      y_view = y_hbm.at[pl.ds(h, vrows)]

      for z in range(3):
        pltpu.sync_copy(zbuf, acc.at[pl.ds(sid * _RPT + z * _ZR, _ZR)])

      @pl.when(sid == _NS - 1)
      def _():
        pltpu.sync_copy(zbuf.at[pl.ds(0, _EXTRA)],
                        acc.at[pl.ds(_EXTRA_START, _EXTRA)])

      plsc.subcore_barrier()

      def ring(j4, carry):
        for k in range(_NBUF):
          off = pl.multiple_of((j4 * _NBUF + k) * _C, 8)

          @pl.when(j4 > 0)
          def _():
            pltpu.make_async_copy(rows[k], acc.at[didx2.at[pl.ds(0, _C)]],
                                  semS[k]).wait()

          pltpu.async_copy(y_view.at[sidx2.at[pl.ds(off, _C)]], rows[k],
                           semG[k])
          if do_cnt:
            pltpu.async_copy(ones, cacc.at[didx2.at[pl.ds(off, _C)]], semC,
                             add=True)
        for k in range(_NBUF):
          off = pl.multiple_of((j4 * _NBUF + k) * _C, 8)
          pltpu.make_async_copy(y_view.at[sidx2.at[pl.ds(off, _C)]], rows[k],
                                semG[k]).wait()
          pltpu.async_copy(rows[k], acc.at[didx2.at[pl.ds(off, _C)]],
                           semS[k], add=True)
        return carry

      lax.fori_loop(0, _NR, ring, 0)
      for k in range(_NBUF):
        pltpu.make_async_copy(rows[k], acc.at[didx2.at[pl.ds(0, _C)]],
                              semS[k]).wait()

      @pl.when(wid < 4)
      def _():
        pltpu.async_copy(y_view.at[sidxe], rows[0].at[pl.ds(0, _XC)],
                         semG[0]).wait()
        pltpu.sync_copy(rows[0].at[pl.ds(0, _XC)], acc.at[didxe], add=True)
        if do_cnt:
          pltpu.sync_copy(ones.at[pl.ds(0, _XC)], cacc.at[didxe], add=True)

      if do_cnt:
        def drain(i, carry):
          pltpu.make_async_copy(ones, cacc.at[didx2.at[pl.ds(0, _C)]],
                                semC).wait()
          return carry
        lax.fori_loop(0, _NCH, drain, 0)

      plsc.subcore_barrier()

      rs = sid * _RPT
      pltpu.sync_copy(acc.at[pl.ds(rs, _RPT)],
                      out_hbm.at[pl.ds(cid * _N + rs, _RPT),
                                 pl.ds(h * _D, _D)])
      if do_cnt:
        pltpu.sync_copy(cacc.at[pl.ds(rs, _RPT)],
                        cnt_hbm.at[pl.ds(cid * _N + rs, _RPT)])

      @pl.when(sid == _NS - 1)
      def _():
        pltpu.sync_copy(acc.at[pl.ds(_EXTRA_START, _EXTRA)],
                        out_hbm.at[pl.ds(cid * _N + _EXTRA_START, _EXTRA),
                                   pl.ds(h * _D, _D)])
        if do_cnt:
          pltpu.sync_copy(cacc.at[pl.ds(_EXTRA_START, _EXTRA)],
                          cnt_hbm.at[pl.ds(cid * _N + _EXTRA_START, _EXTRA)])

      if h + 1 < nparts:
        plsc.subcore_barrier()

  return pl.kernel(
      body, out_type=out_type, mesh=mesh, scratch_types=scratch,
      compiler_params=pltpu.CompilerParams(use_tc_tiling_on_sc=False))


_agg4c = _make_agg(4, True)
_agg4 = _make_agg(4, False)
_agg2 = _make_agg(2, False)


_R = 1000
_G = _N // _R


def _s0_body(x_ref, w_ref, *out_refs, splits):
  res = jnp.dot(x_ref[...], w_ref[...], preferred_element_type=jnp.float32)
  off = 0
  for r, s in zip(out_refs, splits):
    r[...] = res[:, off:off + s]
    off += s


def _stage0(x, wcat, splits):
  K = x.shape[1]
  return pl.pallas_call(
      functools.partial(_s0_body, splits=splits),
      grid=(_G,),
      in_specs=[pl.BlockSpec((_R, K), lambda i: (i, 0)),
                pl.BlockSpec((K, sum(splits)), lambda i: (0, 0))],
      out_specs=[pl.BlockSpec((_R, s), lambda i: (i, 0)) for s in splits],
      out_shape=[jax.ShapeDtypeStruct((_N, s), jnp.float32) for s in splits],
  )(x, wcat)


def _mid_body(parts_ref, cnt_ref, p_ref, b_ref, w_ref, *out_refs, splits):
  cnt = cnt_ref[0] + cnt_ref[1]
  inv = 1.0 / jnp.maximum(cnt, 1.0)
  mean = (parts_ref[0] + parts_ref[1]) * inv
  h = jnp.maximum(mean + b_ref[...] + p_ref[...], 0.0)
  res = jnp.dot(h, w_ref[...], preferred_element_type=jnp.float32)
  off = 0
  for r, s in zip(out_refs, splits):
    r[...] = res[:, off:off + s]
    off += s


def _mid(parts, cnt3, p, b, wcat, splits):
  Din = parts.shape[-1]
  return pl.pallas_call(
      functools.partial(_mid_body, splits=splits),
      grid=(_G,),
      in_specs=[pl.BlockSpec((_NC, _R, Din), lambda i: (0, i, 0)),
                pl.BlockSpec((_NC, _R, 1), lambda i: (0, i, 0)),
                pl.BlockSpec((_R, Din), lambda i: (i, 0)),
                pl.BlockSpec((1, Din), lambda i: (0, 0)),
                pl.BlockSpec((Din, sum(splits)), lambda i: (0, 0))],
      out_specs=[pl.BlockSpec((_R, s), lambda i: (i, 0)) for s in splits],
      out_shape=[jax.ShapeDtypeStruct((_N, s), jnp.float32) for s in splits],
  )(parts, cnt3, p, b, wcat)


def _final_body(parts_ref, cnt_ref, p_ref, b_ref, o_ref):
  cnt = cnt_ref[0] + cnt_ref[1]
  inv = 1.0 / jnp.maximum(cnt, 1.0)
  o = (parts_ref[0] + parts_ref[1]) * inv + b_ref[...] + p_ref[...]
  m = jnp.max(o, axis=1, keepdims=True)
  lse = jnp.log(jnp.sum(jnp.exp(o - m), axis=1, keepdims=True))
  o_ref[...] = o - m - lse


def _final(parts, cnt3, p, b):
  D = parts.shape[-1]
  return pl.pallas_call(
      _final_body,
      grid=(_G,),
      in_specs=[pl.BlockSpec((_NC, _R, D), lambda i: (0, i, 0)),
                pl.BlockSpec((_NC, _R, 1), lambda i: (0, i, 0)),
                pl.BlockSpec((_R, D), lambda i: (i, 0)),
                pl.BlockSpec((1, D), lambda i: (0, 0))],
      out_specs=pl.BlockSpec((_R, D), lambda i: (i, 0)),
      out_shape=jax.ShapeDtypeStruct((_N, D), jnp.float32),
  )(parts, cnt3, p, b)


def kernel(x, edge_index, Wl0, bl0, Wr0, Wl1, bl1, Wr1, Wl2, bl2, Wr2):
  srcr = edge_index[0]
  dstr = edge_index[1]
  w0 = jnp.concatenate([Wl0, Wr0], axis=1)
  w1 = jnp.concatenate([Wl1, Wr1], axis=1)
  w2 = jnp.concatenate([Wl2, Wr2], axis=1)
  b0 = bl0.reshape(1, -1)
  b1 = bl1.reshape(1, -1)
  b2 = bl2.reshape(1, -1)

  y0, p0 = _stage0(x, w0, (128, 128))
  r0 = _agg4c(y0.reshape(4 * _N, _D), srcr, dstr)
  parts0 = r0[0].reshape(_NC, _N, 128)
  cnt3 = r0[1].reshape(_NC, _N, 1)

  y1, p1 = _mid(parts0, cnt3, p0, b0, w1, (128, 128))
  parts1 = _agg4(y1.reshape(4 * _N, _D), srcr, dstr)[0].reshape(_NC, _N, 128)

  y2, p2 = _mid(parts1, cnt3, p1, b1, w2, (64, 64))
  parts2 = _agg2(y2.reshape(2 * _N, _D), srcr, dstr)[0].reshape(_NC, _N, 64)

  return _final(parts2, cnt3, p2, b2)


# speedup vs baseline: 12.5062x; 1.0379x over previous
import functools

import jax
import jax.numpy as jnp
from jax import lax
from jax.experimental import pallas as pl
from jax.experimental.pallas import tpu as pltpu
from jax.experimental.pallas import tpu_sc as plsc

_N = 10000
_E = 320000
_D = 32
_NC = 2
_NS = 16
_NW = _NC * _NS
_EPW = 9984
_C = 416
_NCH = _EPW // _C
_NBUF = 6
_NR = _NCH // _NBUF
_XC = 128
_XBASE = _EPW * _NW
_RPT = 624
_ZR = _RPT // 3
_EXTRA_START = _RPT * _NS
_EXTRA = _N - _EXTRA_START


def _make_agg(nparts, with_cnt):
  mesh = plsc.VectorSubcoreMesh(core_axis_name="c", subcore_axis_name="s")
  shift = nparts.bit_length() - 1
  out_type = [jax.ShapeDtypeStruct((_NC * _N, _D * nparts), jnp.float32)]
  if with_cnt:
    out_type.append(jax.ShapeDtypeStruct((_NC * _N,), jnp.float32))
  scratch = [
      pltpu.VMEM((_EPW,), jnp.int32),
      pltpu.VMEM((_EPW,), jnp.int32),
      pltpu.VMEM((_XC,), jnp.int32),
      pltpu.VMEM((_XC,), jnp.int32),
      *([pltpu.VMEM((_C, _D), jnp.float32)] * _NBUF),
      pltpu.VMEM((_ZR, _D), jnp.float32),
      pltpu.VMEM_SHARED((_N, _D), jnp.float32),
      *([pltpu.SemaphoreType.DMA] * _NBUF),
      *([pltpu.SemaphoreType.DMA] * _NBUF),
  ]
  if with_cnt:
    scratch += [
        pltpu.VMEM((_C,), jnp.float32),
        pltpu.VMEM((_RPT + _EXTRA,), jnp.float32),
        pltpu.VMEM_SHARED((_N,), jnp.float32),
        pltpu.SemaphoreType.DMA,
    ]

  def body(y_hbm, srcr_hbm, dstr_hbm, out_hbm, *rest):
    if with_cnt:
      cnt_hbm = rest[0]
      rest = rest[1:]
    (sidx2, didx2, sidxe, didxe) = rest[:4]
    rows = rest[4:4 + _NBUF]
    zbuf, acc = rest[4 + _NBUF:6 + _NBUF]
    semG = rest[6 + _NBUF:6 + 2 * _NBUF]
    semS = rest[6 + 2 * _NBUF:6 + 3 * _NBUF]
    if with_cnt:
      ones, cbuf, cacc, semC = rest[6 + 3 * _NBUF:]
    cid = lax.axis_index("c")
    sid = lax.axis_index("s")
    wid = sid * _NC + cid

    pltpu.sync_copy(srcr_hbm.at[pl.ds(wid * _EPW, _EPW)], sidx2)
    pltpu.sync_copy(dstr_hbm.at[pl.ds(wid * _EPW, _EPW)], didx2)

    def _scale(i, carry):
      sidx2[pl.ds(i * 16, 16)] = sidx2[pl.ds(i * 16, 16)] << shift
      return carry
    lax.fori_loop(0, _EPW // 16, _scale, 0)

    @pl.when(wid < 4)
    def _():
      xoff = pl.multiple_of(_XBASE + wid * _XC, 8)
      pltpu.sync_copy(srcr_hbm.at[pl.ds(xoff, _XC)], sidxe)
      pltpu.sync_copy(dstr_hbm.at[pl.ds(xoff, _XC)], didxe)

      def _scalee(i, carry):
        sidxe[pl.ds(i * 16, 16)] = sidxe[pl.ds(i * 16, 16)] << shift
        return carry
      lax.fori_loop(0, _XC // 16, _scalee, 0)

    def _zrow(i, carry):
      for k in range(_D // 16):
        zbuf[i, pl.ds(k * 16, 16)] = jnp.zeros((16,), jnp.float32)
      return carry
    lax.fori_loop(0, _ZR, _zrow, 0)

    if with_cnt:
      for k in range(_C // 16):
        ones[pl.ds(k * 16, 16)] = jnp.ones((16,), jnp.float32)
      for k in range((_RPT + _EXTRA) // 16):
        cbuf[pl.ds(k * 16, 16)] = jnp.zeros((16,), jnp.float32)
      pltpu.sync_copy(cbuf.at[pl.ds(0, _RPT)],
                      cacc.at[pl.ds(sid * _RPT, _RPT)])

      @pl.when(sid == _NS - 1)
      def _():
        pltpu.sync_copy(cbuf.at[pl.ds(0, _EXTRA)],
                        cacc.at[pl.ds(_EXTRA_START, _EXTRA)])

    vrows = nparts * _N - (nparts - 1)
    for h in range(nparts):
      do_cnt = with_cnt and h == 0
      y_view = y_hbm.at[pl.ds(h, vrows)]

      for z in range(3):
        pltpu.sync_copy(zbuf, acc.at[pl.ds(sid * _RPT + z * _ZR, _ZR)])

      @pl.when(sid == _NS - 1)
      def _():
        pltpu.sync_copy(zbuf.at[pl.ds(0, _EXTRA)],
                        acc.at[pl.ds(_EXTRA_START, _EXTRA)])

      plsc.subcore_barrier()

      def ring(j4, carry):
        for k in range(_NBUF):
          off = pl.multiple_of((j4 * _NBUF + k) * _C, 8)

          @pl.when(j4 > 0)
          def _():
            pltpu.make_async_copy(rows[k], acc.at[didx2.at[pl.ds(0, _C)]],
                                  semS[k]).wait()

          pltpu.async_copy(y_view.at[sidx2.at[pl.ds(off, _C)]], rows[k],
                           semG[k])
          if do_cnt:
            pltpu.async_copy(ones, cacc.at[didx2.at[pl.ds(off, _C)]], semC,
                             add=True)
        for k in range(_NBUF):
          off = pl.multiple_of((j4 * _NBUF + k) * _C, 8)
          pltpu.make_async_copy(y_view.at[sidx2.at[pl.ds(off, _C)]], rows[k],
                                semG[k]).wait()
          pltpu.async_copy(rows[k], acc.at[didx2.at[pl.ds(off, _C)]],
                           semS[k], add=True)
        return carry

      lax.fori_loop(0, _NR, ring, 0)
      for k in range(_NBUF):
        pltpu.make_async_copy(rows[k], acc.at[didx2.at[pl.ds(0, _C)]],
                              semS[k]).wait()

      @pl.when(wid < 4)
      def _():
        pltpu.async_copy(y_view.at[sidxe], rows[0].at[pl.ds(0, _XC)],
                         semG[0]).wait()
        pltpu.sync_copy(rows[0].at[pl.ds(0, _XC)], acc.at[didxe], add=True)
        if do_cnt:
          pltpu.sync_copy(ones.at[pl.ds(0, _XC)], cacc.at[didxe], add=True)

      if do_cnt:
        def drain(i, carry):
          pltpu.make_async_copy(ones, cacc.at[didx2.at[pl.ds(0, _C)]],
                                semC).wait()
          return carry
        lax.fori_loop(0, _NCH, drain, 0)

      plsc.subcore_barrier()

      rs = sid * _RPT
      pltpu.sync_copy(acc.at[pl.ds(rs, _RPT)],
                      out_hbm.at[pl.ds(cid * _N + rs, _RPT),
                                 pl.ds(h * _D, _D)])
      if do_cnt:
        pltpu.sync_copy(cacc.at[pl.ds(rs, _RPT)],
                        cnt_hbm.at[pl.ds(cid * _N + rs, _RPT)])

      @pl.when(sid == _NS - 1)
      def _():
        pltpu.sync_copy(acc.at[pl.ds(_EXTRA_START, _EXTRA)],
                        out_hbm.at[pl.ds(cid * _N + _EXTRA_START, _EXTRA),
                                   pl.ds(h * _D, _D)])
        if do_cnt:
          pltpu.sync_copy(cacc.at[pl.ds(_EXTRA_START, _EXTRA)],
                          cnt_hbm.at[pl.ds(cid * _N + _EXTRA_START, _EXTRA)])

      if h + 1 < nparts:
        plsc.subcore_barrier()

  return pl.kernel(
      body, out_type=out_type, mesh=mesh, scratch_types=scratch,
      compiler_params=pltpu.CompilerParams(use_tc_tiling_on_sc=False))


_agg4c = _make_agg(4, True)
_agg4 = _make_agg(4, False)
_agg2 = _make_agg(2, False)


_R = 1000
_G = _N // _R


def _s0_body(x_ref, w_ref, *out_refs, splits):
  res = jnp.dot(x_ref[...], w_ref[...], preferred_element_type=jnp.float32)
  off = 0
  for r, s in zip(out_refs, splits):
    r[...] = res[:, off:off + s]
    off += s


def _stage0(x, wcat, splits):
  K = x.shape[1]
  return pl.pallas_call(
      functools.partial(_s0_body, splits=splits),
      grid=(_G,),
      in_specs=[pl.BlockSpec((_R, K), lambda i: (i, 0)),
                pl.BlockSpec((K, sum(splits)), lambda i: (0, 0))],
      out_specs=[pl.BlockSpec((_R, s), lambda i: (i, 0)) for s in splits],
      out_shape=[jax.ShapeDtypeStruct((_N, s), jnp.float32) for s in splits],
  )(x, wcat)


def _mid_body(parts_ref, cnt_ref, p_ref, b_ref, w_ref, *out_refs, splits):
  cnt = cnt_ref[0] + cnt_ref[1]
  inv = 1.0 / jnp.maximum(cnt, 1.0)
  mean = (parts_ref[0] + parts_ref[1]) * inv
  h = jnp.maximum(mean + b_ref[...] + p_ref[...], 0.0)
  res = jnp.dot(h, w_ref[...], preferred_element_type=jnp.float32)
  off = 0
  for r, s in zip(out_refs, splits):
    r[...] = res[:, off:off + s]
    off += s


def _mid(parts, cnt3, p, b, wcat, splits):
  Din = parts.shape[-1]
  return pl.pallas_call(
      functools.partial(_mid_body, splits=splits),
      grid=(_G,),
      in_specs=[pl.BlockSpec((_NC, _R, Din), lambda i: (0, i, 0)),
                pl.BlockSpec((_NC, _R, 1), lambda i: (0, i, 0)),
                pl.BlockSpec((_R, Din), lambda i: (i, 0)),
                pl.BlockSpec((1, Din), lambda i: (0, 0)),
                pl.BlockSpec((Din, sum(splits)), lambda i: (0, 0))],
      out_specs=[pl.BlockSpec((_R, s), lambda i: (i, 0)) for s in splits],
      out_shape=[jax.ShapeDtypeStruct((_N, s), jnp.float32) for s in splits],
  )(parts, cnt3, p, b, wcat)


def _final_body(parts_ref, cnt_ref, p_ref, b_ref, o_ref):
  cnt = cnt_ref[0] + cnt_ref[1]
  inv = 1.0 / jnp.maximum(cnt, 1.0)
  o = (parts_ref[0] + parts_ref[1]) * inv + b_ref[...] + p_ref[...]
  m = jnp.max(o, axis=1, keepdims=True)
  lse = jnp.log(jnp.sum(jnp.exp(o - m), axis=1, keepdims=True))
  o_ref[...] = o - m - lse


def _final(parts, cnt3, p, b):
  D = parts.shape[-1]
  return pl.pallas_call(
      _final_body,
      grid=(_G,),
      in_specs=[pl.BlockSpec((_NC, _R, D), lambda i: (0, i, 0)),
                pl.BlockSpec((_NC, _R, 1), lambda i: (0, i, 0)),
                pl.BlockSpec((_R, D), lambda i: (i, 0)),
                pl.BlockSpec((1, D), lambda i: (0, 0))],
      out_specs=pl.BlockSpec((_R, D), lambda i: (i, 0)),
      out_shape=jax.ShapeDtypeStruct((_N, D), jnp.float32),
  )(parts, cnt3, p, b)


def kernel(x, edge_index, Wl0, bl0, Wr0, Wl1, bl1, Wr1, Wl2, bl2, Wr2):
  srcr = edge_index[0]
  dstr = edge_index[1]
  w0 = jnp.concatenate([Wl0, Wr0], axis=1)
  w1 = jnp.concatenate([Wl1, Wr1], axis=1)
  w2 = jnp.concatenate([Wl2, Wr2], axis=1)
  b0 = bl0.reshape(1, -1)
  b1 = bl1.reshape(1, -1)
  b2 = bl2.reshape(1, -1)

  y0, p0 = _stage0(x, w0, (128, 128))
  r0 = _agg4c(y0.reshape(4 * _N, _D), srcr, dstr)
  parts0 = r0[0].reshape(_NC, _N, 128)
  cnt3 = r0[1].reshape(_NC, _N, 1)

  y1, p1 = _mid(parts0, cnt3, p0, b0, w1, (128, 128))
  parts1 = _agg4(y1.reshape(4 * _N, _D), srcr, dstr)[0].reshape(_NC, _N, 128)

  y2, p2 = _mid(parts1, cnt3, p1, b1, w2, (64, 64))
  parts2 = _agg2(y2.reshape(2 * _N, _D), srcr, dstr)[0].reshape(_NC, _N, 64)

  return _final(parts2, cnt3, p2, b2)
